# Initial kernel scaffold; baseline (speedup 1.0000x reference)
import jax
import jax.numpy as jnp
from jax.experimental import pallas as pl


def kernel(x, table):
    raise NotImplementedError("write your pallas kernel here")


# speedup vs baseline: 1.5758x; 1.5758x over previous
import functools

import jax
import jax.numpy as jnp
from jax import lax
from jax.experimental import pallas as pl
from jax.experimental.pallas import tpu as pltpu
from jax.experimental.pallas import tpu_sc as plsc

_D = 32
_B = 16384 * 26
_NW = 32
_BPW = _B // _NW
_NCHUNK = 13
_C = _BPW // _NCHUNK


def _build():
    mesh = plsc.VectorSubcoreMesh(core_axis_name="c", subcore_axis_name="s")

    @functools.partial(
        pl.kernel,
        mesh=mesh,
        compiler_params=pltpu.CompilerParams(use_tc_tiling_on_sc=False),
        out_type=jax.ShapeDtypeStruct((_B, _D), jnp.float32),
        scratch_types=[
            pltpu.VMEM((_BPW,), jnp.int32),
            pltpu.VMEM((2, _C, _D), jnp.float32),
            pltpu.SemaphoreType.DMA((2,)),
            pltpu.SemaphoreType.DMA((2,)),
        ],
    )
    def k(idx_hbm, table_hbm, out_hbm, idx_v, rows_v, g_sem, s_sem):
        wid = lax.axis_index("s") * 2 + lax.axis_index("c")
        base = wid * _BPW
        pltpu.sync_copy(idx_hbm.at[pl.ds(base, _BPW)], idx_v)

        def gather(c, buf):
            return pltpu.async_copy(
                table_hbm.at[idx_v.at[pl.ds(c * _C, _C)]],
                rows_v.at[buf], g_sem.at[buf])

        def store(c, buf):
            return pltpu.async_copy(
                rows_v.at[buf], out_hbm.at[pl.ds(base + c * _C, _C)],
                s_sem.at[buf])

        g = [None] * _NCHUNK
        s = [None] * _NCHUNK
        g[0] = gather(0, 0)
        for c in range(_NCHUNK):
            buf = c % 2
            if c + 1 < _NCHUNK:
                if c >= 1:
                    s[c - 1].wait()
                g[c + 1] = gather(c + 1, (c + 1) % 2)
            g[c].wait()
            s[c] = store(c, buf)
        s[_NCHUNK - 2].wait()
        s[_NCHUNK - 1].wait()

    return k


_gather_call = _build()


@jax.jit
def kernel(x, table):
    idx = x.reshape(-1)
    out = _gather_call(idx, table)
    return out.reshape(x.shape + (table.shape[1],))
